# baseline (device time: 16511 ns/iter reference)
import jax
import jax.numpy as jnp
from jax import lax
from jax.experimental import pallas as pl
from jax.experimental.pallas import tpu as pltpu

N_DEV = 4
B = 2
S = 256
HQ = 4
DH = 64
BLK = 64
D_MODEL = 512

BF = jnp.bfloat16
F32 = jnp.float32
I8 = jnp.int8
QSCALE = 32.0


def kernel(x, Wq, K_ext, V_ext, Wo):
    wq16 = Wq.astype(BF)
    wo16 = Wo.astype(BF)
    kf = K_ext.reshape(B, S, HQ * DH)
    vf = V_ext.reshape(B, S, HQ * DH)

    def body(x_ref, wq_ref, kf_ref, vf_ref, wo_ref, out_ref,
             k8_stage, v8_stage, k_all, v_all,
             sk_sems, sv_sems, rk_sems, rv_sems):
        my = lax.axis_index("i")

        barrier = pltpu.get_barrier_semaphore()
        for t in range(N_DEV):
            @pl.when(my != t)
            def _():
                pl.semaphore_signal(
                    barrier, inc=1,
                    device_id=(t,), device_id_type=pl.DeviceIdType.MESH,
                )
        pl.semaphore_wait(barrier, N_DEV - 1)

        def quantize(ref):
            return jnp.clip(jnp.round(ref[...] * QSCALE), -127, 127) \
                      .astype(I8)

        k8_stage[...] = quantize(kf_ref)

        for s in range(N_DEV):
            for t in reversed(range(s + 1, N_DEV)):
                @pl.when(my == s)
                def _(s=s, t=t):
                    pltpu.make_async_remote_copy(
                        src_ref=k8_stage, dst_ref=k_all.at[s],
                        send_sem=sk_sems.at[t], recv_sem=rk_sems.at[s],
                        device_id=(t,), device_id_type=pl.DeviceIdType.MESH,
                    ).start()

        v8_stage[...] = quantize(vf_ref)
        for s in range(N_DEV):
            for t in reversed(range(s + 1, N_DEV)):
                @pl.when(my == s)
                def _(s=s, t=t):
                    pltpu.make_async_remote_copy(
                        src_ref=v8_stage, dst_ref=v_all.at[s],
                        send_sem=sv_sems.at[t], recv_sem=rv_sems.at[s],
                        device_id=(t,), device_id_type=pl.DeviceIdType.MESH,
                    ).start()

        q16 = [
            (jnp.dot(x_ref[b].astype(BF), wq_ref[...],
                     preferred_element_type=F32)
             * (0.125 / QSCALE)).astype(BF)
            for b in range(B)
        ]

        q_blk = lax.broadcasted_iota(jnp.int32, (S, S), 0) // BLK
        k_blk = lax.broadcasted_iota(jnp.int32, (S, S), 1) // BLK
        own_mask = k_blk <= q_blk

        def chunk_scores(qh, k_bh, mask_or_vis):
            sc = lax.dot_general(qh, k_bh.astype(BF),
                                 (((1,), (1,)), ((), ())),
                                 preferred_element_type=F32)
            return jnp.exp(jnp.where(mask_or_vis, sc, -1e9))

        l_sum = []
        acc = []
        for b in range(B):
            for h in range(HQ):
                qh = q16[b][:, h * DH:(h + 1) * DH]
                e = chunk_scores(qh, k8_stage[b, :, h * DH:(h + 1) * DH],
                                 own_mask)
                l_sum.append(jnp.sum(e, axis=1, keepdims=True))
                acc.append(jnp.dot(
                    e.astype(BF),
                    v8_stage[b, :, h * DH:(h + 1) * DH].astype(BF),
                    preferred_element_type=F32))

        for s in (2, 0, 1):
            @pl.when(my > s)
            def _(s=s):
                pltpu.make_async_remote_copy(
                    src_ref=k8_stage, dst_ref=k_all.at[s],
                    send_sem=sk_sems.at[s], recv_sem=rk_sems.at[s],
                    device_id=(s,), device_id_type=pl.DeviceIdType.MESH,
                ).wait_recv()

            vis = my > s
            es = []
            i = 0
            for b in range(B):
                for h in range(HQ):
                    qh = q16[b][:, h * DH:(h + 1) * DH]
                    e = chunk_scores(qh, k_all[s, b, :, h * DH:(h + 1) * DH],
                                     vis)
                    l_sum[i] = l_sum[i] + jnp.sum(e, axis=1, keepdims=True)
                    es.append(e.astype(BF))
                    i += 1

            @pl.when(my > s)
            def _(s=s):
                pltpu.make_async_remote_copy(
                    src_ref=v8_stage, dst_ref=v_all.at[s],
                    send_sem=sv_sems.at[s], recv_sem=rv_sems.at[s],
                    device_id=(s,), device_id_type=pl.DeviceIdType.MESH,
                ).wait_recv()

            i = 0
            for b in range(B):
                for h in range(HQ):
                    pv = jnp.dot(
                        es[i],
                        v_all[s, b, :, h * DH:(h + 1) * DH].astype(BF),
                        preferred_element_type=F32)
                    acc[i] = acc[i] + jnp.where(vis, pv, 0.0)
                    i += 1

        for b in range(B):
            ctx = jnp.concatenate(
                [(acc[b * HQ + h] * (1.0 / QSCALE) / l_sum[b * HQ + h])
                 .astype(BF) for h in range(HQ)], axis=1)
            out_ref[b] = jnp.dot(ctx, wo_ref[...],
                                 preferred_element_type=F32)

        for s in range(N_DEV):
            for t in range(s + 1, N_DEV):
                @pl.when(my == s)
                def _(s=s, t=t):
                    pltpu.make_async_remote_copy(
                        src_ref=k8_stage, dst_ref=k_all.at[s],
                        send_sem=sk_sems.at[t], recv_sem=rk_sems.at[s],
                        device_id=(t,), device_id_type=pl.DeviceIdType.MESH,
                    ).wait_send()
                    pltpu.make_async_remote_copy(
                        src_ref=v8_stage, dst_ref=v_all.at[s],
                        send_sem=sv_sems.at[t], recv_sem=rv_sems.at[s],
                        device_id=(t,), device_id_type=pl.DeviceIdType.MESH,
                    ).wait_send()

    return pl.pallas_call(
        body,
        out_shape=jax.ShapeDtypeStruct((B, S, D_MODEL), F32),
        in_specs=[pl.BlockSpec(memory_space=pltpu.VMEM)] * 5,
        out_specs=pl.BlockSpec(memory_space=pltpu.VMEM),
        scratch_shapes=[
            pltpu.VMEM((B, S, HQ * DH), I8),
            pltpu.VMEM((B, S, HQ * DH), I8),
            pltpu.VMEM((N_DEV, B, S, HQ * DH), I8),
            pltpu.VMEM((N_DEV, B, S, HQ * DH), I8),
            pltpu.SemaphoreType.DMA((N_DEV,)),
            pltpu.SemaphoreType.DMA((N_DEV,)),
            pltpu.SemaphoreType.DMA((N_DEV,)),
            pltpu.SemaphoreType.DMA((N_DEV,)),
        ],
        compiler_params=pltpu.CompilerParams(collective_id=0),
    )(x, wq16, kf, vf, wo16)


# device time: 15065 ns/iter; 1.0960x vs baseline; 1.0960x over previous
import jax
import jax.numpy as jnp
from jax import lax
from jax.experimental import pallas as pl
from jax.experimental.pallas import tpu as pltpu

N_DEV = 4
B = 2
S = 256
HQ = 4
DH = 64
BLK = 64
D_MODEL = 512

BF = jnp.bfloat16
F32 = jnp.float32
I8 = jnp.int8
QSCALE = 32.0


def kernel(x, Wq, K_ext, V_ext, Wo):
    wq16 = Wq.astype(BF)
    wo16 = Wo.astype(BF)
    kt8 = jnp.transpose(
        jnp.clip(jnp.round(K_ext * QSCALE), -127, 127).astype(I8),
        (0, 2, 3, 1))
    vt8 = jnp.transpose(
        jnp.clip(jnp.round(V_ext * QSCALE), -127, 127).astype(I8),
        (0, 2, 3, 1))

    def body(x_ref, wq_ref, kt8_ref, vt8_ref, wo_ref,
             out_ref, k_all, v_all, sk_sems, sv_sems, rk_sems, rv_sems):
        my = lax.axis_index("i")

        barrier = pltpu.get_barrier_semaphore()
        for t in range(N_DEV):
            @pl.when(my != t)
            def _():
                pl.semaphore_signal(
                    barrier, inc=1,
                    device_id=(t,), device_id_type=pl.DeviceIdType.MESH,
                )
        pl.semaphore_wait(barrier, N_DEV - 1)

        for s in range(N_DEV):
            for t in reversed(range(s + 1, N_DEV)):
                @pl.when(my == s)
                def _(s=s, t=t):
                    pltpu.make_async_remote_copy(
                        src_ref=kt8_ref, dst_ref=k_all.at[s],
                        send_sem=sk_sems.at[t], recv_sem=rk_sems.at[s],
                        device_id=(t,), device_id_type=pl.DeviceIdType.MESH,
                    ).start()
                    pltpu.make_async_remote_copy(
                        src_ref=vt8_ref, dst_ref=v_all.at[s],
                        send_sem=sv_sems.at[t], recv_sem=rv_sems.at[s],
                        device_id=(t,), device_id_type=pl.DeviceIdType.MESH,
                    ).start()

        q16 = [
            (jnp.dot(x_ref[b].astype(BF), wq_ref[...],
                     preferred_element_type=F32)
             * (0.125 / QSCALE)).astype(BF)
            for b in range(B)
        ]

        q_blk = lax.broadcasted_iota(jnp.int32, (S, S), 0) // BLK
        k_blk = lax.broadcasted_iota(jnp.int32, (S, S), 1) // BLK
        own_mask = k_blk <= q_blk

        l_sum = []
        acc = []
        for b in range(B):
            for h in range(HQ):
                qh = q16[b][:, h * DH:(h + 1) * DH]
                sc = jnp.dot(qh, kt8_ref[b, h].astype(BF),
                             preferred_element_type=F32)
                e = jnp.exp(jnp.where(own_mask, sc, -1e9))
                l_sum.append(jnp.sum(e, axis=1, keepdims=True))
                acc.append(lax.dot_general(
                    e.astype(BF), vt8_ref[b, h].astype(BF),
                    (((1,), (1,)), ((), ())),
                    preferred_element_type=F32))

        for s in (2, 0, 1):
            @pl.when(my > s)
            def _(s=s):
                pltpu.make_async_remote_copy(
                    src_ref=kt8_ref, dst_ref=k_all.at[s],
                    send_sem=sk_sems.at[s], recv_sem=rk_sems.at[s],
                    device_id=(s,), device_id_type=pl.DeviceIdType.MESH,
                ).wait_recv()

            vis = my > s
            es = []
            i = 0
            for b in range(B):
                for h in range(HQ):
                    qh = q16[b][:, h * DH:(h + 1) * DH]
                    sc = jnp.dot(qh, k_all[s, b, h].astype(BF),
                                 preferred_element_type=F32)
                    e = jnp.exp(jnp.where(vis, sc, -1e9))
                    l_sum[i] = l_sum[i] + jnp.sum(e, axis=1, keepdims=True)
                    es.append(e.astype(BF))
                    i += 1

            @pl.when(my > s)
            def _(s=s):
                pltpu.make_async_remote_copy(
                    src_ref=vt8_ref, dst_ref=v_all.at[s],
                    send_sem=sv_sems.at[s], recv_sem=rv_sems.at[s],
                    device_id=(s,), device_id_type=pl.DeviceIdType.MESH,
                ).wait_recv()

            i = 0
            for b in range(B):
                for h in range(HQ):
                    pv = lax.dot_general(
                        es[i], v_all[s, b, h].astype(BF),
                        (((1,), (1,)), ((), ())),
                        preferred_element_type=F32)
                    acc[i] = acc[i] + jnp.where(vis, pv, 0.0)
                    i += 1

        for b in range(B):
            ctx = jnp.concatenate(
                [(acc[b * HQ + h] * (1.0 / QSCALE) / l_sum[b * HQ + h])
                 .astype(BF) for h in range(HQ)], axis=1)
            out_ref[b] = jnp.dot(ctx, wo_ref[...],
                                 preferred_element_type=F32)

        for s in range(N_DEV):
            for t in range(s + 1, N_DEV):
                @pl.when(my == s)
                def _(s=s, t=t):
                    pltpu.make_async_remote_copy(
                        src_ref=kt8_ref, dst_ref=k_all.at[s],
                        send_sem=sk_sems.at[t], recv_sem=rk_sems.at[s],
                        device_id=(t,), device_id_type=pl.DeviceIdType.MESH,
                    ).wait_send()
                    pltpu.make_async_remote_copy(
                        src_ref=vt8_ref, dst_ref=v_all.at[s],
                        send_sem=sv_sems.at[t], recv_sem=rv_sems.at[s],
                        device_id=(t,), device_id_type=pl.DeviceIdType.MESH,
                    ).wait_send()

    return pl.pallas_call(
        body,
        out_shape=jax.ShapeDtypeStruct((B, S, D_MODEL), F32),
        in_specs=[pl.BlockSpec(memory_space=pltpu.VMEM)] * 5,
        out_specs=pl.BlockSpec(memory_space=pltpu.VMEM),
        scratch_shapes=[
            pltpu.VMEM((N_DEV, B, HQ, DH, S), I8),
            pltpu.VMEM((N_DEV, B, HQ, DH, S), I8),
            pltpu.SemaphoreType.DMA((N_DEV,)),
            pltpu.SemaphoreType.DMA((N_DEV,)),
            pltpu.SemaphoreType.DMA((N_DEV,)),
            pltpu.SemaphoreType.DMA((N_DEV,)),
        ],
        compiler_params=pltpu.CompilerParams(collective_id=0),
    )(x, wq16, kt8, vt8, wo16)
